# baseline (speedup 1.0000x reference)
import functools

import jax
import jax.numpy as jnp
from jax import lax
from jax.experimental import pallas as pl
from jax.experimental.pallas import tpu as pltpu
from jax.experimental.pallas import tpu_sc as plsc

B = 8
N_SITES = 2048
N_JUNC = 1024
N_TX = 64
J_PER_TX = 16
D = 256
DP = D // 2
IN_CH = 2 * D
HID = 512

HB = B // 2
BSTEP = 2
HALF_ROWS = 2 * HB * N_JUNC
NW = 32
ROWS_PER_W = HALF_ROWS // NW
CHUNK = 128
N_CHUNKS = ROWS_PER_W // CHUNK


def _sc_gather(table, idx):
    mesh = plsc.VectorSubcoreMesh(core_axis_name="c", subcore_axis_name="s")

    @functools.partial(
        pl.kernel,
        mesh=mesh,
        out_type=jax.ShapeDtypeStruct((HALF_ROWS, DP), jnp.int32),
        scratch_types=[
            pltpu.VMEM((N_CHUNKS, CHUNK), jnp.int32),
            pltpu.VMEM((CHUNK, DP), jnp.int32),
            pltpu.VMEM((CHUNK, DP), jnp.int32),
            pltpu.SemaphoreType.DMA,
            pltpu.SemaphoreType.DMA,
        ],
    )
    def k(table_hbm, idx_hbm, out_hbm, idx_v, buf0, buf1, sem0, sem1):
        wid = lax.axis_index("s") * 2 + lax.axis_index("c")
        base = wid * ROWS_PER_W
        pltpu.sync_copy(idx_hbm.at[wid], idx_v)
        bufs = (buf0, buf1)
        sems = (sem0, sem1)
        prev = pltpu.async_copy(table_hbm.at[idx_v.at[0]], bufs[0], sems[0])
        for c in range(1, N_CHUNKS):
            cur = pltpu.async_copy(table_hbm.at[idx_v.at[c]], bufs[c % 2], sems[c % 2])
            prev.wait()
            pltpu.sync_copy(bufs[(c - 1) % 2],
                            out_hbm.at[pl.ds(base + (c - 1) * CHUNK, CHUNK)])
            prev = cur
        prev.wait()
        pltpu.sync_copy(bufs[(N_CHUNKS - 1) % 2],
                        out_hbm.at[pl.ds(base + (N_CHUNKS - 1) * CHUNK, CHUNK)])

    return k(table, idx)


def _unpack(x):
    lo = lax.bitcast_convert_type(x << 16, jnp.float32).astype(jnp.bfloat16)
    hi = lax.bitcast_convert_type(x & jnp.int32(-65536),
                                  jnp.float32).astype(jnp.bfloat16)
    return lo, hi


def _tc_body(xd_ref, xa_ref, w1_ref, b1_ref, w2_ref, b2_ref, w3_ref,
             ids_ref, scal_ref, out_ref):
    xdl, xdh = _unpack(xd_ref[...])
    xal, xah = _unpack(xa_ref[...])
    h1 = (jnp.dot(xdl, w1_ref[0:DP], preferred_element_type=jnp.float32)
          + jnp.dot(xdh, w1_ref[DP:D], preferred_element_type=jnp.float32)
          + jnp.dot(xal, w1_ref[D:D + DP], preferred_element_type=jnp.float32)
          + jnp.dot(xah, w1_ref[D + DP:], preferred_element_type=jnp.float32))
    h1 = jnp.maximum(h1 + b1_ref[0:1], 0.0).astype(jnp.bfloat16)
    h2 = jnp.dot(h1, w2_ref[...], preferred_element_type=jnp.float32)
    h2 = jnp.maximum(h2 + b2_ref[0:1], 0.0).astype(jnp.bfloat16)
    pot = jnp.dot(h2, w3_ref[...], preferred_element_type=jnp.float32)
    pot = pot + scal_ref[0, 0]

    jidx = lax.broadcasted_iota(jnp.int32, (N_TX, N_JUNC), 1)
    rows = lax.broadcasted_iota(jnp.int32, (128, 128), 0)
    neg_inf = jnp.float32(-jnp.inf)
    for i in range(BSTEP):
        ids = ids_ref[i]
        a = jnp.zeros((N_TX, N_JUNC), jnp.float32)
        for k in range(J_PER_TX):
            a = a + (ids[:, k:k + 1] == jidx).astype(jnp.float32)
        pot_i = pot[i * N_JUNC:(i + 1) * N_JUNC]
        tp = jnp.dot(a, pot_i, preferred_element_type=jnp.float32)
        tp_pad = jnp.concatenate(
            [tp, jnp.zeros((128 - N_TX, 128), jnp.float32)], axis=0)
        v = jnp.where(rows < N_TX, tp_pad,
                      jnp.where(rows == N_TX, scal_ref[0, 1], neg_inf))
        m = jnp.max(v, axis=0, keepdims=True)
        e = jnp.exp(v - m)
        out_ref[i] = e / jnp.sum(e, axis=0, keepdims=True)


def _tc_mlp(embs, w1, b1r, w2, b2r, w3, ids, scal, half):
    return pl.pallas_call(
        _tc_body,
        grid=(HB // BSTEP,),
        in_specs=[
            pl.BlockSpec((BSTEP * N_JUNC, DP), lambda s: (s, 0)),
            pl.BlockSpec((BSTEP * N_JUNC, DP), lambda s: (HB // BSTEP + s, 0)),
            pl.BlockSpec((IN_CH, HID), lambda s: (0, 0)),
            pl.BlockSpec((8, HID), lambda s: (0, 0)),
            pl.BlockSpec((HID, HID), lambda s: (0, 0)),
            pl.BlockSpec((8, HID), lambda s: (0, 0)),
            pl.BlockSpec((HID, 128), lambda s: (0, 0)),
            pl.BlockSpec((BSTEP, N_TX, 128),
                         lambda s, h=half: (h * (HB // BSTEP) + s, 0, 0)),
            pl.BlockSpec((1, 2), lambda s: (0, 0), memory_space=pltpu.SMEM),
        ],
        out_specs=pl.BlockSpec((BSTEP, 128, 128), lambda s: (s, 0, 0)),
        out_shape=jax.ShapeDtypeStruct((HB, 128, 128), jnp.float32),
    )(embs, embs, w1, b1r, w2, b2r, w3, ids, scal)


def kernel(splice_site_reps, junction_indices, transcript_junction_ids,
           W1, b1, W2, b2, W3, b3, ref_potential):
    bits = lax.bitcast_convert_type(splice_site_reps.reshape(B * N_SITES, D),
                                    jnp.int32)
    rne = bits + 0x7FFF + ((bits >> 16) & 1)
    table = (((rne[:, :DP] >> 16) & 0xFFFF)
             | (rne[:, DP:] & jnp.int32(-65536)))
    idx = (junction_indices.astype(jnp.int32)
           + (jnp.arange(B, dtype=jnp.int32) * N_SITES)[:, None, None])
    idx = jnp.transpose(idx, (2, 0, 1))

    w1 = W1.astype(jnp.bfloat16)
    w2 = W2.astype(jnp.bfloat16)
    w3 = jnp.tile(W3, (1, 128)).astype(jnp.bfloat16)
    b1r = jnp.broadcast_to(b1, (8, HID))
    b2r = jnp.broadcast_to(b2, (8, HID))
    scal = jnp.concatenate([b3, ref_potential]).reshape(1, 2)
    ids = jnp.pad(transcript_junction_ids.astype(jnp.int32),
                  ((0, 0), (0, 0), (0, 128 - J_PER_TX)))

    outs = []
    for half in range(2):
        idx_h = idx[:, half * HB:(half + 1) * HB].reshape(NW, N_CHUNKS, CHUNK)
        embs = _sc_gather(table, idx_h)
        outs.append(_tc_mlp(embs, w1, b1r, w2, b2r, w3, ids, scal, half))
    out = jnp.concatenate(outs, axis=0)
    return out[:, :N_TX + 1, 0]


# speedup vs baseline: 1.1716x; 1.1716x over previous
import functools

import jax
import jax.numpy as jnp
from jax import lax
from jax.experimental import pallas as pl
from jax.experimental.pallas import tpu as pltpu
from jax.experimental.pallas import tpu_sc as plsc

B = 8
N_SITES = 2048
N_JUNC = 1024
N_TX = 64
J_PER_TX = 16
D = 256
DP = D // 2
IN_CH = 2 * D
HID = 512

HB = B // 2
BSTEP = 2
HALF_ROWS = 2 * HB * N_JUNC
NW = 32
ROWS_PER_W = HALF_ROWS // NW
CHUNK = 128
N_CHUNKS = ROWS_PER_W // CHUNK


def _sc_gather(table, idx):
    mesh = plsc.VectorSubcoreMesh(core_axis_name="c", subcore_axis_name="s")

    @functools.partial(
        pl.kernel,
        mesh=mesh,
        out_type=jax.ShapeDtypeStruct((HALF_ROWS, D), jnp.float32),
        scratch_types=[
            pltpu.VMEM((N_CHUNKS, CHUNK), jnp.int32),
            pltpu.VMEM((CHUNK, D), jnp.float32),
            pltpu.VMEM((CHUNK, D), jnp.float32),
            pltpu.SemaphoreType.DMA,
            pltpu.SemaphoreType.DMA,
        ],
    )
    def k(table_hbm, idx_hbm, out_hbm, idx_v, buf0, buf1, sem0, sem1):
        wid = lax.axis_index("s") * 2 + lax.axis_index("c")
        base = wid * ROWS_PER_W
        pltpu.sync_copy(idx_hbm.at[wid], idx_v)
        bufs = (buf0, buf1)
        sems = (sem0, sem1)
        prev = pltpu.async_copy(table_hbm.at[idx_v.at[0]], bufs[0], sems[0])
        for c in range(1, N_CHUNKS):
            cur = pltpu.async_copy(table_hbm.at[idx_v.at[c]], bufs[c % 2], sems[c % 2])
            prev.wait()
            pltpu.sync_copy(bufs[(c - 1) % 2],
                            out_hbm.at[pl.ds(base + (c - 1) * CHUNK, CHUNK)])
            prev = cur
        prev.wait()
        pltpu.sync_copy(bufs[(N_CHUNKS - 1) % 2],
                        out_hbm.at[pl.ds(base + (N_CHUNKS - 1) * CHUNK, CHUNK)])

    return k(table, idx)


def _tc_body(xd_ref, xa_ref, w1d_ref, w1a_ref, b1_ref, w2_ref, b2_ref,
             w3_ref, ids_ref, scal_ref, out_ref):
    xd = xd_ref[...].astype(jnp.bfloat16)
    xa = xa_ref[...].astype(jnp.bfloat16)
    h1 = (jnp.dot(xd, w1d_ref[...], preferred_element_type=jnp.float32)
          + jnp.dot(xa, w1a_ref[...], preferred_element_type=jnp.float32))
    h1 = jnp.maximum(h1 + b1_ref[0:1], 0.0).astype(jnp.bfloat16)
    h2 = jnp.dot(h1, w2_ref[...], preferred_element_type=jnp.float32)
    h2 = jnp.maximum(h2 + b2_ref[0:1], 0.0).astype(jnp.bfloat16)

    jidx = lax.broadcasted_iota(jnp.int32, (N_TX, N_JUNC), 1)
    rows = lax.broadcasted_iota(jnp.int32, (128, 128), 0)
    neg_inf = jnp.float32(-jnp.inf)
    for i in range(BSTEP):
        ids = ids_ref[i]
        a = jnp.zeros((N_TX, N_JUNC), jnp.float32)
        for k in range(J_PER_TX):
            a = a + (ids[:, k:k + 1] == jidx).astype(jnp.float32)
        a = a.astype(jnp.bfloat16)
        h2_i = h2[i * N_JUNC:(i + 1) * N_JUNC]
        th = jnp.dot(a, h2_i, preferred_element_type=jnp.float32)
        tp = jnp.dot(th.astype(jnp.bfloat16), w3_ref[...],
                     preferred_element_type=jnp.float32)
        tp = tp + scal_ref[0, 0] * J_PER_TX
        tp_pad = jnp.concatenate(
            [tp, jnp.zeros((128 - N_TX, 128), jnp.float32)], axis=0)
        v = jnp.where(rows < N_TX, tp_pad,
                      jnp.where(rows == N_TX, scal_ref[0, 1], neg_inf))
        m = jnp.max(v, axis=0, keepdims=True)
        e = jnp.exp(v - m)
        out_ref[i] = e / jnp.sum(e, axis=0, keepdims=True)


def _tc_mlp(embs, w1d, w1a, b1r, w2, b2r, w3, ids, scal, half):
    return pl.pallas_call(
        _tc_body,
        grid=(HB // BSTEP,),
        in_specs=[
            pl.BlockSpec((BSTEP * N_JUNC, D), lambda s: (s, 0)),
            pl.BlockSpec((BSTEP * N_JUNC, D), lambda s: (HB // BSTEP + s, 0)),
            pl.BlockSpec((D, HID), lambda s: (0, 0)),
            pl.BlockSpec((D, HID), lambda s: (0, 0)),
            pl.BlockSpec((8, HID), lambda s: (0, 0)),
            pl.BlockSpec((HID, HID), lambda s: (0, 0)),
            pl.BlockSpec((8, HID), lambda s: (0, 0)),
            pl.BlockSpec((HID, 128), lambda s: (0, 0)),
            pl.BlockSpec((BSTEP, N_TX, 128),
                         lambda s, h=half: (h * (HB // BSTEP) + s, 0, 0)),
            pl.BlockSpec((1, 2), lambda s: (0, 0), memory_space=pltpu.SMEM),
        ],
        out_specs=pl.BlockSpec((BSTEP, 128, 128), lambda s: (s, 0, 0)),
        out_shape=jax.ShapeDtypeStruct((HB, 128, 128), jnp.float32),
    )(embs, embs, w1d, w1a, b1r, w2, b2r, w3, ids, scal)


def kernel(splice_site_reps, junction_indices, transcript_junction_ids,
           W1, b1, W2, b2, W3, b3, ref_potential):
    table = splice_site_reps.reshape(B * N_SITES, D)
    idx = (junction_indices.astype(jnp.int32)
           + (jnp.arange(B, dtype=jnp.int32) * N_SITES)[:, None, None])
    idx = jnp.transpose(idx, (2, 0, 1))

    w1d = W1[:D].astype(jnp.bfloat16)
    w1a = W1[D:].astype(jnp.bfloat16)
    w2 = W2.astype(jnp.bfloat16)
    w3 = jnp.tile(W3, (1, 128)).astype(jnp.bfloat16)
    b1r = jnp.broadcast_to(b1, (8, HID))
    b2r = jnp.broadcast_to(b2, (8, HID))
    scal = jnp.concatenate([b3, ref_potential]).reshape(1, 2)
    ids = jnp.pad(transcript_junction_ids.astype(jnp.int32),
                  ((0, 0), (0, 0), (0, 128 - J_PER_TX)))

    outs = []
    for half in range(2):
        idx_h = idx[:, half * HB:(half + 1) * HB].reshape(NW, N_CHUNKS, CHUNK)
        embs = _sc_gather(table, idx_h)
        outs.append(_tc_mlp(embs, w1d, w1a, b1r, w2, b2r, w3, ids, scal, half))
    out = jnp.concatenate(outs, axis=0)
    return out[:, :N_TX + 1, 0]
